# baseline (speedup 1.0000x reference)
import functools

import jax
import jax.numpy as jnp
from jax import lax
from jax.experimental import pallas as pl
from jax.experimental.pallas import tpu as pltpu
from jax.experimental.pallas import tpu_sc as plsc

VOCAB = 1000000
D = 64
R, C = 4096, 200
CA, CB = 128, C - 128
NC, NS = 2, 16
NW = NC * NS
RPW = R // NW
NCHUNK = RPW * 2
NBUF = 6
AHEAD = 4
SLACK = NBUF - AHEAD


def _sc_gather(xa, xb, table):
  mesh = plsc.VectorSubcoreMesh(
      core_axis_name="c", subcore_axis_name="s", num_cores=NC, num_subcores=NS
  )

  @functools.partial(
      pl.kernel,
      out_type=jax.ShapeDtypeStruct((R, C, 128), jnp.float32),
      mesh=mesh,
      scratch_types=[
          pltpu.VMEM((RPW, CA), jnp.int32),
          pltpu.VMEM((RPW, CB), jnp.int32),
          pltpu.VMEM((NBUF, CA, D), jnp.float32),
          pltpu.SemaphoreType.DMA((NBUF,)),
          pltpu.SemaphoreType.DMA((NBUF,)),
      ],
      compiler_params=pltpu.CompilerParams(use_tc_tiling_on_sc=False),
  )
  def k(table_hbm, xa_hbm, xb_hbm, out_hbm, xa_v, xb_v, rows_v, gsem, wsem):
    wid = lax.axis_index("s") * NC + lax.axis_index("c")
    row0 = wid * RPW
    pltpu.sync_copy(xa_hbm.at[pl.ds(row0, RPW)], xa_v)
    pltpu.sync_copy(xb_hbm.at[pl.ds(row0, RPW)], xb_v)

    def g_desc(r, buf, even):
      if even:
        return pltpu.make_async_copy(
            table_hbm.at[xa_v.at[r]], rows_v.at[buf], gsem.at[buf])
      return pltpu.make_async_copy(
          table_hbm.at[xb_v.at[r]],
          rows_v.at[buf, pl.ds(0, CB)],
          gsem.at[buf])

    def w_desc(r, buf, even):
      if even:
        return pltpu.make_async_copy(
            rows_v.at[buf],
            out_hbm.at[row0 + r, pl.ds(0, CA), pl.ds(0, D)],
            wsem.at[buf])
      return pltpu.make_async_copy(
          rows_v.at[buf, pl.ds(0, CB)],
          out_hbm.at[row0 + r, pl.ds(CA, CB), pl.ds(0, D)],
          wsem.at[buf])

    def by_parity(j, buf, mk, act):
      r = lax.div(j, 2)
      even = lax.rem(j, 2) == 0

      @pl.when(even)
      def _():
        act(mk(r, buf, True))

      @pl.when(jnp.logical_not(even))
      def _():
        act(mk(r, buf, False))

    gather = lambda j, buf: by_parity(j, buf, g_desc, lambda d: d.start())
    gather_wait = lambda j, buf: by_parity(j, buf, g_desc, lambda d: d.wait())
    write = lambda j, buf: by_parity(j, buf, w_desc, lambda d: d.start())
    write_wait = lambda j, buf: by_parity(j, buf, w_desc, lambda d: d.wait())

    for b in range(AHEAD):
      gather(b, b)

    def body(j, _):
      g = j + AHEAD
      buf_g = lax.rem(g, NBUF)

      @pl.when(g < NCHUNK)
      def _refill():
        @pl.when(j >= SLACK)
        def _drain():
          write_wait(j - SLACK, buf_g)

        gather(g, buf_g)

      buf = lax.rem(j, NBUF)
      gather_wait(j, buf)
      write(j, buf)
      return _

    lax.fori_loop(0, NCHUNK, body, None)

    for t in range(NBUF):
      j = NCHUNK - NBUF + t
      write_wait(j, j % NBUF)

  return k(table, xa, xb)


def kernel(x, table):
  x = x.astype(jnp.int32)
  out128 = _sc_gather(x[:, :CA], x[:, CA:], table)
  return out128[..., :D]


# speedup vs baseline: 1.0016x; 1.0016x over previous
import functools

import jax
import jax.numpy as jnp
from jax import lax
from jax.experimental import pallas as pl
from jax.experimental.pallas import tpu as pltpu
from jax.experimental.pallas import tpu_sc as plsc

VOCAB = 1000000
D = 64
R, C = 4096, 200
CA, CB = 128, C - 128
NC, NS = 2, 16
NW = NC * NS
RPW = R // NW
NCHUNK = RPW * 2
NBUF = 8
AHEAD = 6
SLACK = NBUF - AHEAD


def _sc_gather(xa, xb, table):
  mesh = plsc.VectorSubcoreMesh(
      core_axis_name="c", subcore_axis_name="s", num_cores=NC, num_subcores=NS
  )

  @functools.partial(
      pl.kernel,
      out_type=jax.ShapeDtypeStruct((R, C, 128), jnp.float32),
      mesh=mesh,
      scratch_types=[
          pltpu.VMEM((RPW, CA), jnp.int32),
          pltpu.VMEM((RPW, CB), jnp.int32),
          pltpu.VMEM((NBUF, CA, D), jnp.float32),
          pltpu.SemaphoreType.DMA((NBUF,)),
          pltpu.SemaphoreType.DMA((NBUF,)),
      ],
      compiler_params=pltpu.CompilerParams(use_tc_tiling_on_sc=False),
  )
  def k(table_hbm, xa_hbm, xb_hbm, out_hbm, xa_v, xb_v, rows_v, gsem, wsem):
    wid = lax.axis_index("s") * NC + lax.axis_index("c")
    row0 = wid * RPW
    pltpu.sync_copy(xa_hbm.at[pl.ds(row0, RPW)], xa_v)
    pltpu.sync_copy(xb_hbm.at[pl.ds(row0, RPW)], xb_v)

    def g_desc(r, buf, even):
      if even:
        return pltpu.make_async_copy(
            table_hbm.at[xa_v.at[r]], rows_v.at[buf], gsem.at[buf])
      return pltpu.make_async_copy(
          table_hbm.at[xb_v.at[r]],
          rows_v.at[buf, pl.ds(0, CB)],
          gsem.at[buf])

    def w_desc(r, buf, even):
      if even:
        return pltpu.make_async_copy(
            rows_v.at[buf],
            out_hbm.at[row0 + r, pl.ds(0, CA), pl.ds(0, D)],
            wsem.at[buf])
      return pltpu.make_async_copy(
          rows_v.at[buf, pl.ds(0, CB)],
          out_hbm.at[row0 + r, pl.ds(CA, CB), pl.ds(0, D)],
          wsem.at[buf])

    def by_parity(j, buf, mk, act):
      r = lax.div(j, 2)
      even = lax.rem(j, 2) == 0

      @pl.when(even)
      def _():
        act(mk(r, buf, True))

      @pl.when(jnp.logical_not(even))
      def _():
        act(mk(r, buf, False))

    gather = lambda j, buf: by_parity(j, buf, g_desc, lambda d: d.start())
    gather_wait = lambda j, buf: by_parity(j, buf, g_desc, lambda d: d.wait())
    write = lambda j, buf: by_parity(j, buf, w_desc, lambda d: d.start())
    write_wait = lambda j, buf: by_parity(j, buf, w_desc, lambda d: d.wait())

    for b in range(AHEAD):
      gather(b, b)

    def body(j, _):
      g = j + AHEAD
      buf_g = lax.rem(g, NBUF)

      @pl.when(g < NCHUNK)
      def _refill():
        @pl.when(j >= SLACK)
        def _drain():
          write_wait(j - SLACK, buf_g)

        gather(g, buf_g)

      buf = lax.rem(j, NBUF)
      gather_wait(j, buf)
      write(j, buf)
      return _

    lax.fori_loop(0, NCHUNK, body, None)

    for t in range(NBUF):
      j = NCHUNK - NBUF + t
      write_wait(j, j % NBUF)

  return k(table, xa, xb)


def kernel(x, table):
  x = x.astype(jnp.int32)
  out128 = _sc_gather(x[:, :CA], x[:, CA:], table)
  return out128[..., :D]
